# baseline (speedup 1.0000x reference)
import jax
import jax.numpy as jnp
import numpy as np
from jax.experimental import pallas as pl

F32 = jnp.float32
NEG = -1e30


def _masks(H, W, P):
    mv = ((np.arange(H * P) % P) < W).astype(np.float32)[:, None]
    idx = np.arange((H + 2) * P + 8)
    a, b = idx // P, idx % P
    me = (((a >= 1) & (a <= H) & (b >= 1) & (b <= W))).astype(np.float32)[:, None]
    return jnp.asarray(mv), jnp.asarray(me)


_MV84, _ = _masks(84, 84, 88)
_MV42, _ME42 = _masks(42, 42, 48)
_MV21, _ME21 = _masks(21, 21, 24)
_MV11, _ME11 = _masks(11, 11, 16)

_SEL_NP = np.zeros((128, 13 * 16), np.float32)
for _t in range(121):
    _SEL_NP[_t, (_t // 11 + 1) * 16 + (_t % 11 + 1)] = 1.0
_SEL = jnp.asarray(_SEL_NP)

_RM = jnp.asarray((np.arange(128) >= 121).astype(np.float32)[:, None] * NEG)

_EYE18 = jnp.asarray(np.eye(18, dtype=np.float32))


def _conv_flat(z, w9, b, H, P, cout_dummy=None):
    L = H * P
    acc = None
    for dy in range(3):
        for dx in range(3):
            o = dy * P + dx
            t = jnp.dot(z[o:o + L, :], w9[dy * 3 + dx],
                        preferred_element_type=F32)
            acc = t if acc is None else acc + t
    return acc + b


def _embed(y, me, P, C):
    za = jnp.zeros((P + 1, C), F32)
    zb = jnp.zeros((P + 7, C), F32)
    return jnp.concatenate([za, y, zb], axis=0) * me


def _resblock(z, w9a, ba, w9b, bb, me, H, P, C):
    r = jnp.maximum(z, 0.0)
    h = _conv_flat(r, w9a, ba, H, P)
    h = jnp.maximum(h, 0.0)
    h2 = _conv_flat(_embed(h, me, P, C), w9b, bb, H, P)
    return z + _embed(h2, me, P, C)


def _poolmax(y, mv, H, P, C, pad_lo):
    ym = jnp.where(mv > 0, y, NEG)
    parts = []
    if pad_lo:
        parts.append(jnp.full((pad_lo * P + pad_lo, C), NEG, F32))
    parts.append(ym)
    parts.append(jnp.full((2 * P + 2, C), NEG, F32))
    ze = jnp.concatenate(parts, axis=0)
    Lm = (H + pad_lo) * P
    m = None
    for a in range(3):
        for b in range(3):
            s = ze[a * P + b: a * P + b + Lm, :]
            m = s if m is None else jnp.maximum(m, s)
    return m


def _k1(z_ref, w_ref, b_ref, mv_ref, out_ref):
    y = _conv_flat(z_ref[...], w_ref[...], b_ref[...], 84, 88)
    out_ref[...] = _poolmax(y, mv_ref[...], 84, 88, 64, 0)


def _mk_res_stage(H, P, C, Co, pad_lo):
    def body(z_ref, wa_ref, ba_ref, wb_ref, bb_ref, wc_ref, bc_ref,
             wd_ref, bd_ref, me_ref, mv_ref, ws_ref, bs_ref, out_ref):
        me = me_ref[...]
        z = _resblock(z_ref[...], wa_ref[...], ba_ref[...], wb_ref[...],
                      bb_ref[...], me, H, P, C)
        z = _resblock(z, wc_ref[...], bc_ref[...], wd_ref[...], bd_ref[...],
                      me, H, P, C)
        y = _conv_flat(z, ws_ref[...], bs_ref[...], H, P)
        out_ref[...] = _poolmax(y, mv_ref[...], H, P, Co, pad_lo)
    return body


def _k4(z_ref, wa_ref, ba_ref, wb_ref, bb_ref, wc_ref, bc_ref, wd_ref,
        bd_ref, me_ref, sel_ref, phi_ref, rm_ref, w1_ref, b1_ref, w2_ref,
        b2_ref, whe_ref, eye_ref, bh_ref, out_ref):
    me = me_ref[...]
    z = _resblock(z_ref[...], wa_ref[...], ba_ref[...], wb_ref[...],
                  bb_ref[...], me, 11, 16, 128)
    z = _resblock(z, wc_ref[...], bc_ref[...], wd_ref[...], bd_ref[...],
                  me, 11, 16, 128)
    xe = jnp.maximum(z[:13 * 16, :], 0.0)
    tokens = jnp.dot(sel_ref[...], xe, preferred_element_type=F32)
    logits = jnp.dot(tokens, phi_ref[...], preferred_element_type=F32)
    lm = logits + rm_ref[...]
    lm = lm - jnp.max(lm, axis=0, keepdims=True)
    el = jnp.exp(lm)
    disp = el / jnp.sum(el, axis=0, keepdims=True)
    slots = jax.lax.dot_general(disp, tokens, (((0,), (0,)), ((), ())),
                                preferred_element_type=F32)
    ys = []
    for e in range(8):
        se = slots[15 * e:15 * e + 15, :]
        h = jnp.maximum(jnp.dot(se, w1_ref[e], preferred_element_type=F32)
                        + b1_ref[e], 0.0)
        ys.append(jnp.dot(h, w2_ref[e], preferred_element_type=F32)
                  + b2_ref[e])
    yall = jnp.concatenate(ys, axis=0)
    cl = logits - jnp.max(logits, axis=1, keepdims=True)
    ec = jnp.exp(cl)
    comb = ec / jnp.sum(ec, axis=1, keepdims=True)
    out = jnp.dot(comb, yall, preferred_element_type=F32)
    eye = eye_ref[...]
    q = bh_ref[...]
    for k in range(18):
        q = q + jnp.sum(out * whe_ref[k]) * eye[k:k + 1, :]
    out_ref[...] = q


def _call(body, out_rows, out_cols, *args):
    return pl.pallas_call(
        body,
        out_shape=jax.ShapeDtypeStruct((out_rows, out_cols), F32),
    )(*args)


def kernel(x, key, params):
    del key
    p = params
    s0, s1, s2 = p['stage0'], p['stage1'], p['stage2']

    def w9(w):
        return w.reshape(9, w.shape[2], w.shape[3])

    def b2d(b):
        return b.reshape(1, -1)

    def emb(t, pw):
        flat = jnp.pad(t, ((1, 1), (1, pw), (0, 0))).reshape(-1, t.shape[2])
        return jnp.pad(flat, ((0, 8), (0, 0)))

    z0 = emb(x, 3)
    mm0 = _call(_k1, 84 * 88, 64, z0, w9(s0['conv_w']), b2d(s0['conv_b']),
                _MV84)
    p0 = mm0.reshape(84, 88, 64)[0:84:2, 0:84:2]

    z1 = emb(p0, 5)
    mm1 = _call(_mk_res_stage(42, 48, 64, 128, 0), 42 * 48, 128, z1,
                w9(s0['b0_c0_w']), b2d(s0['b0_c0_b']),
                w9(s0['b0_c1_w']), b2d(s0['b0_c1_b']),
                w9(s0['b1_c0_w']), b2d(s0['b1_c0_b']),
                w9(s0['b1_c1_w']), b2d(s0['b1_c1_b']),
                _ME42, _MV42, w9(s1['conv_w']), b2d(s1['conv_b']))
    p1 = mm1.reshape(42, 48, 128)[0:42:2, 0:42:2]

    z2 = emb(p1, 2)
    mm2 = _call(_mk_res_stage(21, 24, 128, 128, 1), 22 * 24, 128, z2,
                w9(s1['b0_c0_w']), b2d(s1['b0_c0_b']),
                w9(s1['b0_c1_w']), b2d(s1['b0_c1_b']),
                w9(s1['b1_c0_w']), b2d(s1['b1_c0_b']),
                w9(s1['b1_c1_w']), b2d(s1['b1_c1_b']),
                _ME21, _MV21, w9(s2['conv_w']), b2d(s2['conv_b']))
    p2 = mm2.reshape(22, 24, 128)[0:22:2, 0:22:2]

    z3 = emb(p2, 4)
    phi2 = p['phi'].reshape(128, 120)
    whe = jnp.pad(p['W_head'].reshape(121, 128, 18),
                  ((0, 7), (0, 0), (0, 0))).transpose(2, 0, 1)
    q = _call(_k4, 1, 18, z3,
              w9(s2['b0_c0_w']), b2d(s2['b0_c0_b']),
              w9(s2['b0_c1_w']), b2d(s2['b0_c1_b']),
              w9(s2['b1_c0_w']), b2d(s2['b1_c0_b']),
              w9(s2['b1_c1_w']), b2d(s2['b1_c1_b']),
              _ME11, _SEL, phi2, _RM,
              p['W1'], p['b1'].reshape(8, 1, 512),
              p['W2'], p['b2'].reshape(8, 1, 128),
              whe, _EYE18, p['b_head'].reshape(1, 18))
    return q.reshape(18)
